# baseline (speedup 1.0000x reference)
import jax
import jax.numpy as jnp
from jax import lax
from jax.experimental import pallas as pl
from jax.experimental.pallas import tpu as pltpu
from jax.experimental.pallas import tpu_sc as plsc

VOCAB = 1000000
EMB = 32
BATCH = 16384
HIST = 200

NC = 2
NS = 16
NW = NC * NS
NBLK = BATCH // 128
BLK_PER_W = NBLK // NW


def _emb_body(table_hbm, seqt_hbm, out_hbm, idx_v, rows_v, ptile_v,
              sem_g, sem_out, sem_idx):
    wid = lax.axis_index("s") * NC + lax.axis_index("c")
    c_iota = lax.iota(jnp.int32, 16)
    c3a = c_iota >> 3
    c7a = c_iota & 7
    c3b = c3a + 2
    zv = c_iota & 0

    def gather(t, d):
        return pltpu.async_copy(table_hbm.at[idx_v.at[t]], rows_v.at[d],
                                sem_g.at[d])

    for a in range(BLK_PER_W):
        blk = wid * BLK_PER_W + a
        b0 = blk * 128

        pltpu.make_async_copy(
            seqt_hbm.at[:, pl.ds(b0, 128)], idx_v, sem_idx).start()
        pltpu.make_async_copy(
            seqt_hbm.at[:, pl.ds(b0, 128)], idx_v, sem_idx).wait()

        gather(0, 0)
        gather(1, 1)

        def outer(t2, carry, a=a, blk=blk):
            for d in range(2):
                t = t2 * 2 + d

                def out_copy(tt, d=d, blk=blk):
                    return pltpu.make_async_copy(
                        ptile_v.at[d],
                        out_hbm.at[tt, :, pl.ds(blk, 1)],
                        sem_out.at[d])

                pltpu.make_async_copy(
                    table_hbm.at[idx_v.at[t]], rows_v.at[d],
                    sem_g.at[d]).wait()

                if a == 0:
                    @pl.when(t >= 2)
                    def _():
                        out_copy(t).wait()
                else:
                    out_copy(t).wait()

                def transpose(j, carry2, d=d):
                    r0 = rows_v[d, j, pl.ds(0, 16)]
                    r1 = rows_v[d, j, pl.ds(16, 16)]
                    jv = zv + j
                    plsc.store_scatter(ptile_v.at[d], [c3a, zv, c7a, jv], r0)
                    plsc.store_scatter(ptile_v.at[d], [c3b, zv, c7a, jv], r1)
                    return carry2

                lax.fori_loop(0, 128, transpose, 0)

                @pl.when(t + 2 < HIST)
                def _():
                    gather(t + 2, d)

                out_copy(t).start()
            return carry

        lax.fori_loop(0, HIST // 2, outer, 0)
    for d in range(2):
        pltpu.make_async_copy(
            ptile_v.at[d], out_hbm.at[0, :, pl.ds(0, 1)],
            sem_out.at[d]).wait()


def kernel(seq, W):
    seqt = seq.T
    mesh = plsc.VectorSubcoreMesh(core_axis_name="c", subcore_axis_name="s")
    f = pl.kernel(
        _emb_body,
        out_type=jax.ShapeDtypeStruct((HIST, 4, NBLK, 8, 128), jnp.float32),
        mesh=mesh,
        scratch_types=[
            pltpu.VMEM((HIST, 128), jnp.int32),
            pltpu.VMEM((2, 128, EMB), jnp.float32),
            pltpu.VMEM((2, 4, 1, 8, 128), jnp.float32),
            pltpu.SemaphoreType.DMA((2,)),
            pltpu.SemaphoreType.DMA((2,)),
            pltpu.SemaphoreType.DMA,
        ],
        compiler_params=pltpu.CompilerParams(
            use_tc_tiling_on_sc=False, needs_layout_passes=False),
    )
    p5 = f(W, seqt)
    return p5.transpose(2, 4, 0, 1, 3).reshape(BATCH, HIST, EMB)
